# baseline (speedup 1.0000x reference)
import functools

import jax
import jax.numpy as jnp
from jax import lax
from jax.experimental import pallas as pl
from jax.experimental.pallas import tpu as pltpu
from jax.experimental.pallas import tpu_sc as plsc

N = 10000
E = 320000
D = 128
G = 64
NPAD = 10240
CH = 80
EPAD = 327680
NCHUNK = EPAD // CH
P1C = NCHUNK // 16
P2C = NCHUNK // 32
F32 = jnp.float32
I32 = jnp.int32


def _lrelu(v):
    return jnp.where(v >= 0, v, 0.2 * v)


NHALF = NPAD // 2
ACCR = NHALF + 128
ROWS_T = ACCR // 16


def _sc_body(asrc_hbm, adst_hbm, src_hbm, dst_hbm, h_hbm, out_hbm,
             asrc_v, adst_v, denom_v, src2_v, dst2_v,
             rows_v, rows2_v, coef_v, coef2_v, dloc_v, dloc2_v,
             exbuf_v, exbuf2_v, zbuf_v, denom_sh, acc_sh,
             sem, sem2):
    c = lax.axis_index("c")
    s = lax.axis_index("s")
    lo = c * NHALF

    pltpu.sync_copy(asrc_hbm, asrc_v)
    pltpu.sync_copy(adst_hbm, adst_v)

    def mbody(i, carry):
        ms, md = carry
        ms = jnp.maximum(ms, asrc_v[pl.ds(i * 16, 16)])
        md = jnp.maximum(md, adst_v[pl.ds(i * 16, 16)])
        return (ms, md)

    init = jnp.full((16,), -3.0e38, F32)
    ms, md = lax.fori_loop(0, NPAD // 16, mbody, (init, init))

    iot = lax.broadcasted_iota(I32, (16,), 0)

    def _lanemax(v):
        for sft in (1, 2, 4, 8):
            exbuf_v[pl.ds(0, 16)] = v
            v = jnp.maximum(v, plsc.load_gather(exbuf_v, [iot ^ sft]))
        return v

    M = _lrelu(_lanemax(ms) + _lanemax(md))

    def zrow(i, _):
        for r in range(8):
            rows_v[i, pl.ds(r * 16, 16)] = jnp.zeros((16,), F32)
        return 0
    lax.fori_loop(0, CH, zrow, 0)

    def zb(i, _):
        zbuf_v[pl.ds(i * 16, 16)] = jnp.zeros((16,), F32)
        return 0
    lax.fori_loop(0, 40, zb, 0)

    pltpu.sync_copy(zbuf_v, denom_sh.at[pl.ds(s * 640, 640)])
    for k in range(4):
        pltpu.sync_copy(rows_v, acc_sh.at[pl.ds(s * ROWS_T + k * CH, CH), :])
    pltpu.sync_copy(rows_v.at[pl.ds(0, 8), :],
                    acc_sh.at[pl.ds(s * ROWS_T + 4 * CH, 8), :])
    plsc.subcore_barrier()

    def _ex_chunk(j, buf):
        for i in range(CH // 16):
            sidx = src2_v[j, pl.ds(i * 16, 16)]
            didx = dst2_v[j, pl.ds(i * 16, 16)]
            a = (plsc.load_gather(asrc_v, [sidx]) +
                 plsc.load_gather(adst_v, [didx]))
            buf[pl.ds(i * 16, 16)] = jnp.exp(_lrelu(a) - M)

    def _drain_ex():
        pltpu.make_async_copy(asrc_hbm.at[pl.ds(0, CH)], exbuf_v, sem2).wait()

    def p1(jj, _):
        j0 = jj * 2
        @pl.when(jj > 0)
        def _():
            _drain_ex()
            _drain_ex()
        _ex_chunk(j0, exbuf_v)
        pltpu.async_copy(exbuf_v, denom_sh.at[dst2_v.at[j0]], sem2, add=True)
        _ex_chunk(j0 + 1, exbuf2_v)
        pltpu.async_copy(exbuf2_v, denom_sh.at[dst2_v.at[j0 + 1]], sem2,
                         add=True)
        return 0

    for seg in range(2):
        pltpu.sync_copy(src_hbm.at[pl.ds((s * 2 + seg) * P2C, P2C)], src2_v)
        pltpu.sync_copy(dst_hbm.at[pl.ds((s * 2 + seg) * P2C, P2C)], dst2_v)
        lax.fori_loop(0, P2C // 2, p1, 0)
        _drain_ex()
        _drain_ex()
    plsc.subcore_barrier()

    pltpu.sync_copy(denom_sh, denom_v)

    def _coef_chunk(j, coefb, dlocb):
        for i in range(CH // 16):
            sidx = src2_v[j, pl.ds(i * 16, 16)]
            didx = dst2_v[j, pl.ds(i * 16, 16)]
            a = (plsc.load_gather(asrc_v, [sidx]) +
                 plsc.load_gather(adst_v, [didx]))
            e = jnp.exp(_lrelu(a) - M)
            dg = plsc.load_gather(denom_v, [didx])
            coefb[pl.ds(i * 16, 16)] = e / (dg + 1e-16)
            inr = (didx >= lo) & (didx < lo + NHALF)
            dlocb[pl.ds(i * 16, 16)] = jnp.where(
                inr, didx - lo, NHALF + s)

    def _scale(rowsb, coefb):
        def body(i, _):
            i0 = i * 2
            ca = plsc.load_gather(coefb, [jnp.zeros((16,), I32) + i0])
            cb = plsc.load_gather(coefb, [jnp.zeros((16,), I32) + i0 + 1])
            for r in range(8):
                rowsb[i0, pl.ds(r * 16, 16)] = (
                    rowsb[i0, pl.ds(r * 16, 16)] * ca)
                rowsb[i0 + 1, pl.ds(r * 16, 16)] = (
                    rowsb[i0 + 1, pl.ds(r * 16, 16)] * cb)
            return 0
        lax.fori_loop(0, CH // 2, body, 0)

    def _drain_rows():
        pltpu.make_async_copy(h_hbm.at[pl.ds(0, CH)], rows_v, sem2).wait()

    def p2(jj, _):
        j0 = jj * 2
        @pl.when(jj > 0)
        def _():
            _drain_rows()
            _drain_rows()
        ga = pltpu.async_copy(h_hbm.at[src2_v.at[j0]], rows_v, sem)
        gb = pltpu.async_copy(h_hbm.at[src2_v.at[j0 + 1]], rows2_v, sem)
        _coef_chunk(j0, coef_v, dloc_v)
        _coef_chunk(j0 + 1, coef2_v, dloc2_v)
        ga.wait()
        _scale(rows_v, coef_v)
        pltpu.async_copy(rows_v, acc_sh.at[dloc_v], sem2, add=True)
        gb.wait()
        _scale(rows2_v, coef2_v)
        pltpu.async_copy(rows2_v, acc_sh.at[dloc2_v], sem2, add=True)
        return 0

    for seg in range(2):
        pltpu.sync_copy(src_hbm.at[pl.ds((s * 2 + seg) * P2C, P2C)], src2_v)
        pltpu.sync_copy(dst_hbm.at[pl.ds((s * 2 + seg) * P2C, P2C)], dst2_v)
        lax.fori_loop(0, P2C // 2, p2, 0)
        _drain_rows()
        _drain_rows()
    plsc.subcore_barrier()

    pltpu.sync_copy(acc_sh.at[pl.ds(s * 320, 320), :],
                    out_hbm.at[pl.ds(c * NHALF + s * 320, 320), :])


_sc_layer = functools.partial(
    pl.kernel,
    out_type=jax.ShapeDtypeStruct((NPAD, D), F32),
    mesh=plsc.VectorSubcoreMesh(core_axis_name="c", subcore_axis_name="s"),
    compiler_params=pltpu.CompilerParams(needs_layout_passes=False),
    scratch_types=[
        pltpu.VMEM((NPAD,), F32),
        pltpu.VMEM((NPAD,), F32),
        pltpu.VMEM((NPAD,), F32),
        pltpu.VMEM((P2C, CH), I32),
        pltpu.VMEM((P2C, CH), I32),
        pltpu.VMEM((CH, D), F32),
        pltpu.VMEM((CH, D), F32),
        pltpu.VMEM((CH,), F32),
        pltpu.VMEM((CH,), F32),
        pltpu.VMEM((CH,), I32),
        pltpu.VMEM((CH,), I32),
        pltpu.VMEM((CH,), F32),
        pltpu.VMEM((CH,), F32),
        pltpu.VMEM((640,), F32),
        pltpu.VMEM_SHARED((NPAD,), F32),
        pltpu.VMEM_SHARED((ACCR, D), F32),
        pltpu.SemaphoreType.DMA,
        pltpu.SemaphoreType.DMA,
    ],
)(_sc_body)


def _pre_body(x_ref, w_ref, asw_ref, adw_ref, h_ref, a1_ref, a2_ref):
    h = jnp.dot(x_ref[...], w_ref[...], preferred_element_type=F32)
    h_ref[...] = h
    pad = jnp.zeros((NPAD - N, 1), F32)
    a1_ref[pl.ds(0, N), :] = jnp.dot(h, asw_ref[...], preferred_element_type=F32,
                                     precision=lax.Precision.HIGHEST)
    a1_ref[pl.ds(N, NPAD - N), :] = pad
    a2_ref[pl.ds(0, N), :] = jnp.dot(h, adw_ref[...], preferred_element_type=F32,
                                     precision=lax.Precision.HIGHEST)
    a2_ref[pl.ds(N, NPAD - N), :] = pad


_tc_pre = pl.pallas_call(
    _pre_body,
    out_shape=[
        jax.ShapeDtypeStruct((N, D), F32),
        jax.ShapeDtypeStruct((NPAD, 1), F32),
        jax.ShapeDtypeStruct((NPAD, 1), F32),
    ],
)


def _mid_body(acc_ref, b_ref, w_ref, asw_ref, adw_ref,
              h_ref, a1_ref, a2_ref):
    g = jnp.maximum(acc_ref[...] + b_ref[...], 0.0)
    h = jnp.dot(g, w_ref[...], preferred_element_type=F32)
    h_ref[...] = h
    pad = jnp.zeros((NPAD - N, 1), F32)
    a1_ref[pl.ds(0, N), :] = jnp.dot(h, asw_ref[...], preferred_element_type=F32,
                                     precision=lax.Precision.HIGHEST)
    a1_ref[pl.ds(N, NPAD - N), :] = pad
    a2_ref[pl.ds(0, N), :] = jnp.dot(h, adw_ref[...], preferred_element_type=F32,
                                     precision=lax.Precision.HIGHEST)
    a2_ref[pl.ds(N, NPAD - N), :] = pad


_tc_mid = pl.pallas_call(
    _mid_body,
    out_shape=[
        jax.ShapeDtypeStruct((N, D), F32),
        jax.ShapeDtypeStruct((NPAD, 1), F32),
        jax.ShapeDtypeStruct((NPAD, 1), F32),
    ],
)


def _fin_body(acc_ref, b_ref, batch_ref, wc_ref, bc_ref, out_ref):
    g = jnp.maximum(acc_ref[...] + b_ref[...], 0.0)
    iota = lax.broadcasted_iota(I32, (G, N), 0)
    oh = (iota == batch_ref[...]).astype(F32)
    sums = jnp.dot(oh, g, preferred_element_type=F32,
                   precision=lax.Precision.HIGHEST)
    counts = jnp.sum(oh, axis=1, keepdims=True)
    pooled = sums / jnp.maximum(counts, 1.0)
    out_ref[...] = (jnp.dot(pooled, wc_ref[...], preferred_element_type=F32)
                    + bc_ref[...])


_tc_fin = pl.pallas_call(
    _fin_body,
    out_shape=jax.ShapeDtypeStruct((G, 2), F32),
)


def kernel(x, edge_index, batch, W1, a_src1, a_dst1, b1,
           W2, a_src2, a_dst2, b2, Wc, bc):
    pad_src = jnp.zeros((EPAD - E,), I32)
    pad_dst = jnp.full((EPAD - E,), N, I32)
    src_r = jnp.concatenate([edge_index[0], pad_src]).reshape(NCHUNK, CH)
    dst_r = jnp.concatenate([edge_index[1], pad_dst]).reshape(NCHUNK, CH)

    h1, as1, ad1 = _tc_pre(x, W1, a_src1.reshape(D, 1), a_dst1.reshape(D, 1))
    acc1 = _sc_layer(as1.reshape(NPAD), ad1.reshape(NPAD), src_r, dst_r, h1)

    h2, as2, ad2 = _tc_mid(acc1[:N, :], b1.reshape(1, D), W2,
                           a_src2.reshape(D, 1), a_dst2.reshape(D, 1))
    acc2 = _sc_layer(as2.reshape(NPAD), ad2.reshape(NPAD), src_r, dst_r, h2)

    return _tc_fin(acc2[:N, :], b2.reshape(1, D),
                   batch.reshape(1, N), Wc, bc.reshape(1, 2))


# speedup vs baseline: 1.0096x; 1.0096x over previous
import functools

import jax
import jax.numpy as jnp
from jax import lax
from jax.experimental import pallas as pl
from jax.experimental.pallas import tpu as pltpu
from jax.experimental.pallas import tpu_sc as plsc

N = 10000
E = 320000
D = 128
G = 64
NPAD = 10240
CH = 80
EPAD = 327680
NCHUNK = EPAD // CH
P1C = NCHUNK // 16
P2C = NCHUNK // 32
F32 = jnp.float32
I32 = jnp.int32


def _lrelu(v):
    return jnp.where(v >= 0, v, 0.2 * v)


NHALF = NPAD // 2
ACCR = NHALF + 128
ROWS_T = ACCR // 16


def _sc_body(asrc_hbm, adst_hbm, src_hbm, dst_hbm, h_hbm, out_hbm,
             asrc_v, adst_v, denom_v, src2_v, dst2_v,
             rows_v, rows2_v, coef_v, coef2_v, dloc_v, dloc2_v,
             exbuf_v, exbuf2_v, zbuf_v, denom_sh, acc_sh,
             sem, sem2):
    c = lax.axis_index("c")
    s = lax.axis_index("s")
    lo = c * NHALF

    pltpu.sync_copy(asrc_hbm, asrc_v)
    pltpu.sync_copy(adst_hbm, adst_v)

    def mbody(i, carry):
        ms, md = carry
        ms = jnp.maximum(ms, asrc_v[pl.ds(i * 16, 16)])
        md = jnp.maximum(md, adst_v[pl.ds(i * 16, 16)])
        return (ms, md)

    init = jnp.full((16,), -3.0e38, F32)
    ms, md = lax.fori_loop(0, NPAD // 16, mbody, (init, init))

    iot = lax.broadcasted_iota(I32, (16,), 0)

    def _lanemax(v):
        for sft in (1, 2, 4, 8):
            exbuf_v[pl.ds(0, 16)] = v
            v = jnp.maximum(v, plsc.load_gather(exbuf_v, [iot ^ sft]))
        return v

    M = _lrelu(_lanemax(ms) + _lanemax(md))

    def zrow(i, _):
        for r in range(8):
            rows_v[i, pl.ds(r * 16, 16)] = jnp.zeros((16,), F32)
        return 0
    lax.fori_loop(0, CH, zrow, 0)

    def zb(i, _):
        zbuf_v[pl.ds(i * 16, 16)] = jnp.zeros((16,), F32)
        return 0
    lax.fori_loop(0, 40, zb, 0)

    pltpu.sync_copy(zbuf_v, denom_sh.at[pl.ds(s * 640, 640)])
    for k in range(4):
        pltpu.sync_copy(rows_v, acc_sh.at[pl.ds(s * ROWS_T + k * CH, CH), :])
    pltpu.sync_copy(rows_v.at[pl.ds(0, 8), :],
                    acc_sh.at[pl.ds(s * ROWS_T + 4 * CH, 8), :])
    plsc.subcore_barrier()

    def _ex_chunk(j, buf):
        for i in range(CH // 16):
            sidx = src2_v[j, pl.ds(i * 16, 16)]
            didx = dst2_v[j, pl.ds(i * 16, 16)]
            a = (plsc.load_gather(asrc_v, [sidx]) +
                 plsc.load_gather(adst_v, [didx]))
            buf[pl.ds(i * 16, 16)] = jnp.exp(_lrelu(a) - M)

    def p1(jj, _):
        j0 = jj * 2
        _ex_chunk(j0, exbuf_v)
        cpa = pltpu.async_copy(exbuf_v, denom_sh.at[dst2_v.at[j0]], sem2,
                               add=True)
        _ex_chunk(j0 + 1, exbuf2_v)
        cpb = pltpu.async_copy(exbuf2_v, denom_sh.at[dst2_v.at[j0 + 1]],
                               sem2, add=True)
        cpa.wait()
        cpb.wait()
        return 0

    for seg in range(2):
        pltpu.sync_copy(src_hbm.at[pl.ds((s * 2 + seg) * P2C, P2C)], src2_v)
        pltpu.sync_copy(dst_hbm.at[pl.ds((s * 2 + seg) * P2C, P2C)], dst2_v)
        lax.fori_loop(0, P2C // 2, p1, 0)
    plsc.subcore_barrier()

    pltpu.sync_copy(denom_sh, denom_v)

    def _coef_chunk(j, coefb, dlocb):
        for i in range(CH // 16):
            sidx = src2_v[j, pl.ds(i * 16, 16)]
            didx = dst2_v[j, pl.ds(i * 16, 16)]
            a = (plsc.load_gather(asrc_v, [sidx]) +
                 plsc.load_gather(adst_v, [didx]))
            e = jnp.exp(_lrelu(a) - M)
            dg = plsc.load_gather(denom_v, [didx])
            coefb[pl.ds(i * 16, 16)] = e / (dg + 1e-16)
            inr = (didx >= lo) & (didx < lo + NHALF)
            dlocb[pl.ds(i * 16, 16)] = jnp.where(
                inr, didx - lo, NHALF + s)

    def _scale(rowsb, coefb):
        def body(i, _):
            i0 = i * 4
            cs = [plsc.load_gather(coefb, [jnp.zeros((16,), I32) + i0 + k])
                  for k in range(4)]
            for r in range(8):
                for k in range(4):
                    rowsb[i0 + k, pl.ds(r * 16, 16)] = (
                        rowsb[i0 + k, pl.ds(r * 16, 16)] * cs[k])
            return 0
        lax.fori_loop(0, CH // 4, body, 0)

    def p2(jj, _):
        j0 = jj * 2
        ga = pltpu.async_copy(h_hbm.at[src2_v.at[j0]], rows_v, sem)
        gb = pltpu.async_copy(h_hbm.at[src2_v.at[j0 + 1]], rows2_v, sem)
        _coef_chunk(j0, coef_v, dloc_v)
        _coef_chunk(j0 + 1, coef2_v, dloc2_v)
        ga.wait()
        _scale(rows_v, coef_v)
        ca = pltpu.async_copy(rows_v, acc_sh.at[dloc_v], sem2, add=True)
        gb.wait()
        _scale(rows2_v, coef2_v)
        cb = pltpu.async_copy(rows2_v, acc_sh.at[dloc2_v], sem2, add=True)
        ca.wait()
        cb.wait()
        return 0

    for seg in range(2):
        pltpu.sync_copy(src_hbm.at[pl.ds((s * 2 + seg) * P2C, P2C)], src2_v)
        pltpu.sync_copy(dst_hbm.at[pl.ds((s * 2 + seg) * P2C, P2C)], dst2_v)
        lax.fori_loop(0, P2C // 2, p2, 0)
    plsc.subcore_barrier()

    pltpu.sync_copy(acc_sh.at[pl.ds(s * 320, 320), :],
                    out_hbm.at[pl.ds(c * NHALF + s * 320, 320), :])


_sc_layer = functools.partial(
    pl.kernel,
    out_type=jax.ShapeDtypeStruct((NPAD, D), F32),
    mesh=plsc.VectorSubcoreMesh(core_axis_name="c", subcore_axis_name="s"),
    compiler_params=pltpu.CompilerParams(needs_layout_passes=False),
    scratch_types=[
        pltpu.VMEM((NPAD,), F32),
        pltpu.VMEM((NPAD,), F32),
        pltpu.VMEM((NPAD,), F32),
        pltpu.VMEM((P2C, CH), I32),
        pltpu.VMEM((P2C, CH), I32),
        pltpu.VMEM((CH, D), F32),
        pltpu.VMEM((CH, D), F32),
        pltpu.VMEM((CH,), F32),
        pltpu.VMEM((CH,), F32),
        pltpu.VMEM((CH,), I32),
        pltpu.VMEM((CH,), I32),
        pltpu.VMEM((CH,), F32),
        pltpu.VMEM((CH,), F32),
        pltpu.VMEM((640,), F32),
        pltpu.VMEM_SHARED((NPAD,), F32),
        pltpu.VMEM_SHARED((ACCR, D), F32),
        pltpu.SemaphoreType.DMA,
        pltpu.SemaphoreType.DMA,
    ],
)(_sc_body)


def _pre_body(x_ref, w_ref, asw_ref, adw_ref, h_ref, a1_ref, a2_ref):
    h = jnp.dot(x_ref[...], w_ref[...], preferred_element_type=F32)
    h_ref[...] = h
    pad = jnp.zeros((NPAD - N, 1), F32)
    a1_ref[pl.ds(0, N), :] = jnp.dot(h, asw_ref[...], preferred_element_type=F32,
                                     precision=lax.Precision.HIGHEST)
    a1_ref[pl.ds(N, NPAD - N), :] = pad
    a2_ref[pl.ds(0, N), :] = jnp.dot(h, adw_ref[...], preferred_element_type=F32,
                                     precision=lax.Precision.HIGHEST)
    a2_ref[pl.ds(N, NPAD - N), :] = pad


_tc_pre = pl.pallas_call(
    _pre_body,
    out_shape=[
        jax.ShapeDtypeStruct((N, D), F32),
        jax.ShapeDtypeStruct((NPAD, 1), F32),
        jax.ShapeDtypeStruct((NPAD, 1), F32),
    ],
)


def _mid_body(acc_ref, b_ref, w_ref, asw_ref, adw_ref,
              h_ref, a1_ref, a2_ref):
    g = jnp.maximum(acc_ref[...] + b_ref[...], 0.0)
    h = jnp.dot(g, w_ref[...], preferred_element_type=F32)
    h_ref[...] = h
    pad = jnp.zeros((NPAD - N, 1), F32)
    a1_ref[pl.ds(0, N), :] = jnp.dot(h, asw_ref[...], preferred_element_type=F32,
                                     precision=lax.Precision.HIGHEST)
    a1_ref[pl.ds(N, NPAD - N), :] = pad
    a2_ref[pl.ds(0, N), :] = jnp.dot(h, adw_ref[...], preferred_element_type=F32,
                                     precision=lax.Precision.HIGHEST)
    a2_ref[pl.ds(N, NPAD - N), :] = pad


_tc_mid = pl.pallas_call(
    _mid_body,
    out_shape=[
        jax.ShapeDtypeStruct((N, D), F32),
        jax.ShapeDtypeStruct((NPAD, 1), F32),
        jax.ShapeDtypeStruct((NPAD, 1), F32),
    ],
)


def _fin_body(acc_ref, b_ref, batch_ref, wc_ref, bc_ref, out_ref):
    g = jnp.maximum(acc_ref[...] + b_ref[...], 0.0)
    iota = lax.broadcasted_iota(I32, (G, N), 0)
    oh = (iota == batch_ref[...]).astype(F32)
    sums = jnp.dot(oh, g, preferred_element_type=F32,
                   precision=lax.Precision.HIGHEST)
    counts = jnp.sum(oh, axis=1, keepdims=True)
    pooled = sums / jnp.maximum(counts, 1.0)
    out_ref[...] = (jnp.dot(pooled, wc_ref[...], preferred_element_type=F32)
                    + bc_ref[...])


_tc_fin = pl.pallas_call(
    _fin_body,
    out_shape=jax.ShapeDtypeStruct((G, 2), F32),
)


def kernel(x, edge_index, batch, W1, a_src1, a_dst1, b1,
           W2, a_src2, a_dst2, b2, Wc, bc):
    pad_src = jnp.zeros((EPAD - E,), I32)
    pad_dst = jnp.full((EPAD - E,), N, I32)
    src_r = jnp.concatenate([edge_index[0], pad_src]).reshape(NCHUNK, CH)
    dst_r = jnp.concatenate([edge_index[1], pad_dst]).reshape(NCHUNK, CH)

    h1, as1, ad1 = _tc_pre(x, W1, a_src1.reshape(D, 1), a_dst1.reshape(D, 1))
    acc1 = _sc_layer(as1.reshape(NPAD), ad1.reshape(NPAD), src_r, dst_r, h1)

    h2, as2, ad2 = _tc_mid(acc1[:N, :], b1.reshape(1, D), W2,
                           a_src2.reshape(D, 1), a_dst2.reshape(D, 1))
    acc2 = _sc_layer(as2.reshape(NPAD), ad2.reshape(NPAD), src_r, dst_r, h2)

    return _tc_fin(acc2[:N, :], b2.reshape(1, D),
                   batch.reshape(1, N), Wc, bc.reshape(1, 2))


# speedup vs baseline: 1.0275x; 1.0177x over previous
import functools

import jax
import jax.numpy as jnp
from jax import lax
from jax.experimental import pallas as pl
from jax.experimental.pallas import tpu as pltpu
from jax.experimental.pallas import tpu_sc as plsc

N = 10000
E = 320000
D = 128
G = 64
NPAD = 10240
CH = 128
EPAD = 327680
NCHUNK = EPAD // CH
TILC = NCHUNK // 16
NSEG = 4
SEGC = TILC // NSEG
F32 = jnp.float32
I32 = jnp.int32


def _lrelu(v):
    return jnp.where(v >= 0, v, 0.2 * v)


NHALF = NPAD // 2
ACCR = NHALF + 128
ROWS_T = ACCR // 16


def _sc_body(asrc_hbm, adst_hbm, src_hbm, dst_hbm, h_hbm, out_hbm,
             asrc_v, adst_v, denom_v, src2_v, dst2_v,
             rows_v, rows2_v, coef_v, coef2_v, dloc_v, dloc2_v,
             exbuf_v, exbuf2_v, zbuf_v, denom_sh, acc_sh,
             sem, sem2):
    c = lax.axis_index("c")
    s = lax.axis_index("s")
    lo = c * NHALF

    pltpu.sync_copy(asrc_hbm, asrc_v)
    pltpu.sync_copy(adst_hbm, adst_v)

    def mbody(i, carry):
        ms, md = carry
        ms = jnp.maximum(ms, asrc_v[pl.ds(i * 16, 16)])
        md = jnp.maximum(md, adst_v[pl.ds(i * 16, 16)])
        return (ms, md)

    init = jnp.full((16,), -3.0e38, F32)
    ms, md = lax.fori_loop(0, NPAD // 16, mbody, (init, init))

    iot = lax.broadcasted_iota(I32, (16,), 0)

    def _lanemax(v):
        for sft in (1, 2, 4, 8):
            exbuf_v[pl.ds(0, 16)] = v
            v = jnp.maximum(v, plsc.load_gather(exbuf_v, [iot ^ sft]))
        return v

    M = _lrelu(_lanemax(ms) + _lanemax(md))

    def zrow(i, _):
        for r in range(8):
            rows_v[i, pl.ds(r * 16, 16)] = jnp.zeros((16,), F32)
        return 0
    lax.fori_loop(0, CH, zrow, 0)

    def zb(i, _):
        zbuf_v[pl.ds(i * 16, 16)] = jnp.zeros((16,), F32)
        return 0
    lax.fori_loop(0, 40, zb, 0)

    pltpu.sync_copy(zbuf_v, denom_sh.at[pl.ds(s * 640, 640)])
    for k in range(2):
        pltpu.sync_copy(rows_v, acc_sh.at[pl.ds(s * ROWS_T + k * CH, CH), :])
    pltpu.sync_copy(rows_v.at[pl.ds(0, 72), :],
                    acc_sh.at[pl.ds(s * ROWS_T + 2 * CH, 72), :])
    plsc.subcore_barrier()

    def _ex_chunk(j, buf):
        for i in range(CH // 16):
            sidx = src2_v[j, pl.ds(i * 16, 16)]
            didx = dst2_v[j, pl.ds(i * 16, 16)]
            a = (plsc.load_gather(asrc_v, [sidx]) +
                 plsc.load_gather(adst_v, [didx]))
            buf[pl.ds(i * 16, 16)] = jnp.exp(_lrelu(a) - M)

    def p1(jj, _):
        j0 = jj * 2
        _ex_chunk(j0, exbuf_v)
        cpa = pltpu.async_copy(exbuf_v, denom_sh.at[dst2_v.at[j0]], sem2,
                               add=True)
        _ex_chunk(j0 + 1, exbuf2_v)
        cpb = pltpu.async_copy(exbuf2_v, denom_sh.at[dst2_v.at[j0 + 1]],
                               sem2, add=True)
        cpa.wait()
        cpb.wait()
        return 0

    for seg in range(NSEG):
        pltpu.sync_copy(src_hbm.at[pl.ds((s * NSEG + seg) * SEGC, SEGC)],
                        src2_v)
        pltpu.sync_copy(dst_hbm.at[pl.ds((s * NSEG + seg) * SEGC, SEGC)],
                        dst2_v)
        lax.fori_loop(0, SEGC // 2, p1, 0)
    plsc.subcore_barrier()

    pltpu.sync_copy(denom_sh, denom_v)

    def _coef_chunk(j, coefb, dlocb):
        for i in range(CH // 16):
            sidx = src2_v[j, pl.ds(i * 16, 16)]
            didx = dst2_v[j, pl.ds(i * 16, 16)]
            a = (plsc.load_gather(asrc_v, [sidx]) +
                 plsc.load_gather(adst_v, [didx]))
            e = jnp.exp(_lrelu(a) - M)
            dg = plsc.load_gather(denom_v, [didx])
            coefb[pl.ds(i * 16, 16)] = e / (dg + 1e-16)
            inr = (didx >= lo) & (didx < lo + NHALF)
            dlocb[pl.ds(i * 16, 16)] = jnp.where(
                inr, didx - lo, NHALF + s)

    def _scale(rowsb, coefb):
        def body(i, _):
            i0 = i * 4
            cs = [plsc.load_gather(coefb, [jnp.zeros((16,), I32) + i0 + k])
                  for k in range(4)]
            for r in range(8):
                for k in range(4):
                    rowsb[i0 + k, pl.ds(r * 16, 16)] = (
                        rowsb[i0 + k, pl.ds(r * 16, 16)] * cs[k])
            return 0
        lax.fori_loop(0, CH // 4, body, 0)

    def p2(jj, _):
        j0 = jj * 2
        ga = pltpu.async_copy(h_hbm.at[src2_v.at[j0]], rows_v, sem)
        gb = pltpu.async_copy(h_hbm.at[src2_v.at[j0 + 1]], rows2_v, sem)
        _coef_chunk(j0, coef_v, dloc_v)
        _coef_chunk(j0 + 1, coef2_v, dloc2_v)
        ga.wait()
        _scale(rows_v, coef_v)
        ca = pltpu.async_copy(rows_v, acc_sh.at[dloc_v], sem2, add=True)
        gb.wait()
        _scale(rows2_v, coef2_v)
        cb = pltpu.async_copy(rows2_v, acc_sh.at[dloc2_v], sem2, add=True)
        ca.wait()
        cb.wait()
        return 0

    for seg in range(NSEG):
        pltpu.sync_copy(src_hbm.at[pl.ds((s * NSEG + seg) * SEGC, SEGC)],
                        src2_v)
        pltpu.sync_copy(dst_hbm.at[pl.ds((s * NSEG + seg) * SEGC, SEGC)],
                        dst2_v)
        lax.fori_loop(0, SEGC // 2, p2, 0)
    plsc.subcore_barrier()

    pltpu.sync_copy(acc_sh.at[pl.ds(s * 320, 320), :],
                    out_hbm.at[pl.ds(c * NHALF + s * 320, 320), :])


_sc_layer = functools.partial(
    pl.kernel,
    out_type=jax.ShapeDtypeStruct((NPAD, D), F32),
    mesh=plsc.VectorSubcoreMesh(core_axis_name="c", subcore_axis_name="s"),
    compiler_params=pltpu.CompilerParams(needs_layout_passes=False),
    scratch_types=[
        pltpu.VMEM((NPAD,), F32),
        pltpu.VMEM((NPAD,), F32),
        pltpu.VMEM((NPAD,), F32),
        pltpu.VMEM((SEGC, CH), I32),
        pltpu.VMEM((SEGC, CH), I32),
        pltpu.VMEM((CH, D), F32),
        pltpu.VMEM((CH, D), F32),
        pltpu.VMEM((CH,), F32),
        pltpu.VMEM((CH,), F32),
        pltpu.VMEM((CH,), I32),
        pltpu.VMEM((CH,), I32),
        pltpu.VMEM((CH,), F32),
        pltpu.VMEM((CH,), F32),
        pltpu.VMEM((640,), F32),
        pltpu.VMEM_SHARED((NPAD,), F32),
        pltpu.VMEM_SHARED((ACCR, D), F32),
        pltpu.SemaphoreType.DMA,
        pltpu.SemaphoreType.DMA,
    ],
)(_sc_body)


def _pre_body(x_ref, w_ref, asw_ref, adw_ref, h_ref, a1_ref, a2_ref):
    h = jnp.dot(x_ref[...], w_ref[...], preferred_element_type=F32)
    h_ref[...] = h
    pad = jnp.zeros((NPAD - N, 1), F32)
    a1_ref[pl.ds(0, N), :] = jnp.dot(h, asw_ref[...], preferred_element_type=F32,
                                     precision=lax.Precision.HIGHEST)
    a1_ref[pl.ds(N, NPAD - N), :] = pad
    a2_ref[pl.ds(0, N), :] = jnp.dot(h, adw_ref[...], preferred_element_type=F32,
                                     precision=lax.Precision.HIGHEST)
    a2_ref[pl.ds(N, NPAD - N), :] = pad


_tc_pre = pl.pallas_call(
    _pre_body,
    out_shape=[
        jax.ShapeDtypeStruct((N, D), F32),
        jax.ShapeDtypeStruct((NPAD, 1), F32),
        jax.ShapeDtypeStruct((NPAD, 1), F32),
    ],
)


def _mid_body(acc_ref, b_ref, w_ref, asw_ref, adw_ref,
              h_ref, a1_ref, a2_ref):
    g = jnp.maximum(acc_ref[...] + b_ref[...], 0.0)
    h = jnp.dot(g, w_ref[...], preferred_element_type=F32)
    h_ref[...] = h
    pad = jnp.zeros((NPAD - N, 1), F32)
    a1_ref[pl.ds(0, N), :] = jnp.dot(h, asw_ref[...], preferred_element_type=F32,
                                     precision=lax.Precision.HIGHEST)
    a1_ref[pl.ds(N, NPAD - N), :] = pad
    a2_ref[pl.ds(0, N), :] = jnp.dot(h, adw_ref[...], preferred_element_type=F32,
                                     precision=lax.Precision.HIGHEST)
    a2_ref[pl.ds(N, NPAD - N), :] = pad


_tc_mid = pl.pallas_call(
    _mid_body,
    out_shape=[
        jax.ShapeDtypeStruct((N, D), F32),
        jax.ShapeDtypeStruct((NPAD, 1), F32),
        jax.ShapeDtypeStruct((NPAD, 1), F32),
    ],
)


def _fin_body(acc_ref, b_ref, batch_ref, wc_ref, bc_ref, out_ref):
    g = jnp.maximum(acc_ref[...] + b_ref[...], 0.0)
    iota = lax.broadcasted_iota(I32, (G, N), 0)
    oh = (iota == batch_ref[...]).astype(F32)
    sums = jnp.dot(oh, g, preferred_element_type=F32,
                   precision=lax.Precision.HIGHEST)
    counts = jnp.sum(oh, axis=1, keepdims=True)
    pooled = sums / jnp.maximum(counts, 1.0)
    out_ref[...] = (jnp.dot(pooled, wc_ref[...], preferred_element_type=F32)
                    + bc_ref[...])


_tc_fin = pl.pallas_call(
    _fin_body,
    out_shape=jax.ShapeDtypeStruct((G, 2), F32),
)


def kernel(x, edge_index, batch, W1, a_src1, a_dst1, b1,
           W2, a_src2, a_dst2, b2, Wc, bc):
    pad_src = jnp.zeros((EPAD - E,), I32)
    pad_dst = jnp.full((EPAD - E,), N, I32)
    src_r = jnp.concatenate([edge_index[0], pad_src]).reshape(NCHUNK, CH)
    dst_r = jnp.concatenate([edge_index[1], pad_dst]).reshape(NCHUNK, CH)

    h1, as1, ad1 = _tc_pre(x, W1, a_src1.reshape(D, 1), a_dst1.reshape(D, 1))
    acc1 = _sc_layer(as1.reshape(NPAD), ad1.reshape(NPAD), src_r, dst_r, h1)

    h2, as2, ad2 = _tc_mid(acc1[:N, :], b1.reshape(1, D), W2,
                           a_src2.reshape(D, 1), a_dst2.reshape(D, 1))
    acc2 = _sc_layer(as2.reshape(NPAD), ad2.reshape(NPAD), src_r, dst_r, h2)

    return _tc_fin(acc2[:N, :], b2.reshape(1, D),
                   batch.reshape(1, N), Wc, bc.reshape(1, 2))
